# Initial kernel scaffold; baseline (speedup 1.0000x reference)
import functools

import jax
import jax.numpy as jnp
from jax import lax
from jax.experimental import pallas as pl
from jax.experimental.pallas import tpu as pltpu
from jax.experimental.pallas import tpu_sc as plsc

B = 1024
L = 200
V = 100000
D = 64

NC = 2
NS = 16
NW = NC * NS
NBL = B * L
PW = NBL // NW
C = 128
NCH = PW // C

_MESH = plsc.VectorSubcoreMesh(core_axis_name="c", subcore_axis_name="s")


def _sc_body(tok_hbm, typ_hbm, ttab_hbm, seg_hbm, pos_hbm, out_hbm,
             idx_v, typ_v, seg_v, pos_v, comb_v, rows_v, gsem):
  wid = lax.axis_index("c") * NS + lax.axis_index("s")

  pltpu.sync_copy(tok_hbm.at[wid], idx_v)
  pltpu.sync_copy(typ_hbm.at[wid], typ_v)
  pltpu.sync_copy(seg_hbm, seg_v)
  pltpu.sync_copy(pos_hbm, pos_v)

  def build(l, carry):
    for j in range(D // 16):
      sl = pl.ds(j * 16, 16)
      p = pos_v[l, sl]
      comb_v[l, sl] = p + seg_v[0, sl]
      comb_v[L + l, sl] = p + seg_v[1, sl]
    return carry
  lax.fori_loop(0, L, build, 0)

  def chunk(c, carry):
    base = wid * PW + c * C
    pltpu.async_copy(ttab_hbm.at[idx_v.at[c]], rows_v, gsem).wait()

    def row(r, rcarry):
      off = c * C + r
      l = lax.rem(off, L)
      t = typ_v[off]
      ci = t * L + l
      for j in range(D // 16):
        sl = pl.ds(j * 16, 16)
        plsc.addupdate(rows_v.at[r, sl], comb_v[ci, sl])
      return rcarry
    lax.fori_loop(0, C, row, 0)

    pltpu.sync_copy(rows_v, out_hbm.at[pl.ds(base, C)])
    return carry
  lax.fori_loop(0, NCH, chunk, 0)


@jax.jit
def _sc_embed(tok3, typ2, ttab, seg, pos):
  return pl.kernel(
      _sc_body,
      out_type=jax.ShapeDtypeStruct((NBL, D), jnp.float32),
      mesh=_MESH,
      scratch_types=[
          pltpu.VMEM((NCH, C), jnp.int32),
          pltpu.VMEM((PW,), jnp.int32),
          pltpu.VMEM((2, D), jnp.float32),
          pltpu.VMEM((L, D), jnp.float32),
          pltpu.VMEM((2 * L, D), jnp.float32),
          pltpu.VMEM((C, D), jnp.float32),
          pltpu.SemaphoreType.DMA,
      ],
  )(tok3, typ2, ttab, seg, pos)


def _mask_body(ids_ref, out_ref):
  out_ref[...] = ids_ref[...] != 0


@jax.jit
def _mask_call(token_ids):
  return pl.pallas_call(
      _mask_body,
      out_shape=jax.ShapeDtypeStruct((B, L), jnp.bool_),
  )(token_ids)


def kernel(token_ids, type_token_ids, token_table, segment_table, positional_table):
  tok3 = token_ids.astype(jnp.int32).reshape(NW, NCH, C)
  typ2 = type_token_ids.astype(jnp.int32).reshape(NW, PW)
  out = _sc_embed(tok3, typ2, token_table, segment_table, positional_table)
  outputs = out.reshape(B, L, D)
  attention_mask = _mask_call(token_ids).reshape(B, 1, 1, L)
  return outputs, attention_mask


# speedup vs baseline: 3.8868x; 3.8868x over previous
import functools

import jax
import jax.numpy as jnp
from jax import lax
from jax.experimental import pallas as pl
from jax.experimental.pallas import tpu as pltpu
from jax.experimental.pallas import tpu_sc as plsc

B = 1024
L = 200
V = 100000
D = 64

NC = 2
NS = 16
NW = NC * NS
NBL = B * L
PW = NBL // NW
C = 128
NCH = PW // C

def _sc_body(tok_hbm, typ_hbm, ttab_hbm, seg_hbm, pos_hbm, out_hbm,
             idx_v, typ_v, seg_v, pos_v, comb_v, rows_v, gsem):
  wid = lax.axis_index("c") * NS + lax.axis_index("s")

  pltpu.sync_copy(tok_hbm.at[wid], idx_v)
  pltpu.sync_copy(typ_hbm.at[wid], typ_v)
  pltpu.sync_copy(seg_hbm, seg_v)
  pltpu.sync_copy(pos_hbm, pos_v)

  def build(l, carry):
    for j in range(D // 16):
      sl = pl.ds(j * 16, 16)
      p = pos_v[l, sl]
      comb_v[l, sl] = p + seg_v[0, sl]
      comb_v[L + l, sl] = p + seg_v[1, sl]
    return carry
  lax.fori_loop(0, L, build, 0)

  def cib(i, carry):
    sl = pl.ds(i * 16, 16)
    lv = lax.rem(i * 16 + lax.iota(jnp.int32, 16), L)
    typ_v[sl] = typ_v[sl] * L + lv
    return carry
  lax.fori_loop(0, PW // 16, cib, 0)

  def chunk(c, carry):
    base = wid * PW + c * C
    pltpu.async_copy(ttab_hbm.at[idx_v.at[c]], rows_v, gsem).wait()

    def grp(g, gcarry):
      civ = typ_v[pl.ds(c * C + g * 16, 16)]
      r0 = g * 16
      for k in range(16):
        ci = civ[k]
        for j in range(D // 16):
          sl = pl.ds(j * 16, 16)
          plsc.addupdate(rows_v.at[r0 + k, sl], comb_v[ci, sl])
      return gcarry
    lax.fori_loop(0, C // 16, grp, 0)

    pltpu.sync_copy(rows_v, out_hbm.at[pl.ds(base, C)])
    return carry
  lax.fori_loop(0, NCH, chunk, 0)


@jax.jit
def _sc_embed(tok3, typ2, ttab, seg, pos):
  return pl.kernel(
      _sc_body,
      out_type=jax.ShapeDtypeStruct((NBL, D), jnp.float32),
      mesh=plsc.VectorSubcoreMesh(core_axis_name="c", subcore_axis_name="s"),
      compiler_params=pltpu.CompilerParams(use_tc_tiling_on_sc=False),
      scratch_types=[
          pltpu.VMEM((NCH, C), jnp.int32),
          pltpu.VMEM((PW,), jnp.int32),
          pltpu.VMEM((2, D), jnp.float32),
          pltpu.VMEM((L, D), jnp.float32),
          pltpu.VMEM((2 * L, D), jnp.float32),
          pltpu.VMEM((C, D), jnp.float32),
          pltpu.SemaphoreType.DMA,
      ],
  )(tok3, typ2, ttab, seg, pos)


def _mask_body(ids_ref, out_ref):
  out_ref[...] = ids_ref[...] != 0


@jax.jit
def _mask_call(token_ids):
  return pl.pallas_call(
      _mask_body,
      out_shape=jax.ShapeDtypeStruct((B, L), jnp.bool_),
  )(token_ids)


def kernel(token_ids, type_token_ids, token_table, segment_table, positional_table):
  tok3 = token_ids.astype(jnp.int32).reshape(NW, NCH, C)
  typ2 = type_token_ids.astype(jnp.int32).reshape(NW, PW)
  out = _sc_embed(tok3, typ2, token_table, segment_table, positional_table)
  outputs = out.reshape(B, L, D)
  attention_mask = _mask_call(token_ids).reshape(B, 1, 1, L)
  return outputs, attention_mask
